# Initial kernel scaffold; baseline (speedup 1.0000x reference)
import jax
import jax.numpy as jnp
from jax.experimental import pallas as pl


def kernel(x, gamma1, beta1, gamma2, beta2, Wr, br, W1e, b1e, W2e, b2e, Wd1, bd1, Wd2, bd2, Wg, bg):
    raise NotImplementedError("write your pallas kernel here")


# speedup vs baseline: 3.2845x; 3.2845x over previous
import jax
import jax.numpy as jnp
from jax.experimental import pallas as pl
from jax.experimental.pallas import tpu as pltpu

T = 2048
D = 1024
H = 4096
S = 8
EPS = 1e-5

BH = 1024
NH = H // BH
BT2 = 512
NT2 = T // BT2

_DOT = jnp.float32


def _layer_norm(x, g, b):
    mu = jnp.mean(x, axis=-1, keepdims=True)
    var = jnp.mean((x - mu) ** 2, axis=-1, keepdims=True)
    return (x - mu) * jax.lax.rsqrt(var + EPS) * g + b


def _gelu(x):
    return 0.5 * x * (1.0 + jax.lax.erf(x * 0.7071067811865476))


def _router_kernel(x_ref, g1_ref, b1_ref, wr_ref, br_ref, normed_ref, fw_ref):
    x = x_ref[...]
    normed = _layer_norm(x, g1_ref[...], b1_ref[...])
    normed_ref[...] = normed.astype(normed_ref.dtype)
    logits = jnp.dot(normed, wr_ref[...], preferred_element_type=jnp.float32)
    logits = logits + br_ref[...]
    iota = jax.lax.broadcasted_iota(jnp.int32, logits.shape, 1)
    v1 = jnp.max(logits, axis=-1, keepdims=True)
    i1 = jnp.min(jnp.where(logits == v1, iota, S), axis=-1, keepdims=True)
    l2 = jnp.where(iota == i1, -jnp.inf, logits)
    v2 = jnp.max(l2, axis=-1, keepdims=True)
    i2 = jnp.min(jnp.where(l2 == v2, iota, S), axis=-1, keepdims=True)
    e2 = jnp.exp(v2 - v1)
    w1 = 1.0 / (1.0 + e2)
    w2 = e2 * w1
    fw_ref[...] = jnp.where(iota == i1, w1, 0.0) + jnp.where(iota == i2, w2, 0.0)


def _slot_kernel(normed_ref, fw_ref, w1_ref, b1_ref, w2_ref, b2_ref, out_ref):
    s = pl.program_id(0)
    hb = pl.program_id(1)
    iota = jax.lax.broadcasted_iota(jnp.int32, (T, S), 1)
    w_col = jnp.sum(jnp.where(iota == s, fw_ref[...], 0.0), axis=-1,
                    keepdims=True)
    normed = normed_ref[...]
    h1 = jnp.dot(normed, w1_ref[0], preferred_element_type=jnp.float32)
    h1 = h1 + b1_ref[0]
    g = _gelu(h1).astype(_DOT)
    y = jnp.dot(g, w2_ref[0], preferred_element_type=jnp.float32)
    contrib = w_col * y

    @pl.when(jnp.logical_and(s == 0, hb == 0))
    def _init():
        out_ref[...] = jnp.zeros_like(out_ref)

    @pl.when(hb == 0)
    def _bias():
        out_ref[...] += w_col * b2_ref[0]

    out_ref[...] += contrib


def _dense_kernel(x_ref, so_ref, g2_ref, b2_ref, wd1_ref, bd1_ref, wd2_ref,
                  bd2_ref, wg_ref, bg_ref, out_ref):
    x = x_ref[...]
    so = so_ref[...]
    x1 = x + so
    x1n = _layer_norm(x1, g2_ref[...], b2_ref[...])
    gate_logit = jnp.sum(x1n * wg_ref[...], axis=-1, keepdims=True) + bg_ref[0, 0]
    gate = jax.nn.sigmoid(gate_logit)
    h = jnp.dot(x1n.astype(_DOT), wd1_ref[...],
                preferred_element_type=jnp.float32) + bd1_ref[...]
    g = _gelu(h).astype(_DOT)
    do = jnp.dot(g, wd2_ref[...], preferred_element_type=jnp.float32)
    do = do + bd2_ref[...]
    out_ref[0] = x1 + gate * so + (1.0 - gate) * do


def kernel(x, gamma1, beta1, gamma2, beta2, Wr, br, W1e, b1e, W2e, b2e,
           Wd1, bd1, Wd2, bd2, Wg, bg):
    x2d = x.reshape(T, D)

    normed, fw = pl.pallas_call(
        _router_kernel,
        out_shape=(
            jax.ShapeDtypeStruct((T, D), _DOT),
            jax.ShapeDtypeStruct((T, S), jnp.float32),
        ),
    )(x2d, gamma1.reshape(1, D), beta1.reshape(1, D), Wr, br.reshape(1, S))

    slot_out = pl.pallas_call(
        _slot_kernel,
        grid=(S, NH),
        in_specs=[
            pl.BlockSpec((T, D), lambda s, hb: (0, 0)),
            pl.BlockSpec((T, S), lambda s, hb: (0, 0)),
            pl.BlockSpec((1, D, BH), lambda s, hb: (s, 0, hb)),
            pl.BlockSpec((1, 1, BH), lambda s, hb: (s, 0, hb)),
            pl.BlockSpec((1, BH, D), lambda s, hb: (s, hb, 0)),
            pl.BlockSpec((1, 1, D), lambda s, hb: (s, 0, 0)),
        ],
        out_specs=pl.BlockSpec((T, D), lambda s, hb: (0, 0)),
        out_shape=jax.ShapeDtypeStruct((T, D), jnp.float32),
        compiler_params=pltpu.CompilerParams(
            dimension_semantics=("arbitrary", "arbitrary"),
        ),
    )(
        normed,
        fw,
        W1e.astype(_DOT),
        b1e.reshape(S, 1, H),
        W2e.astype(_DOT),
        b2e.reshape(S, 1, D),
    )

    out = pl.pallas_call(
        _dense_kernel,
        grid=(NT2,),
        in_specs=[
            pl.BlockSpec((BT2, D), lambda t: (t, 0)),
            pl.BlockSpec((BT2, D), lambda t: (t, 0)),
            pl.BlockSpec((1, D), lambda t: (0, 0)),
            pl.BlockSpec((1, D), lambda t: (0, 0)),
            pl.BlockSpec((D, H), lambda t: (0, 0)),
            pl.BlockSpec((1, H), lambda t: (0, 0)),
            pl.BlockSpec((H, D), lambda t: (0, 0)),
            pl.BlockSpec((1, D), lambda t: (0, 0)),
            pl.BlockSpec((1, D), lambda t: (0, 0)),
            pl.BlockSpec((1, 1), lambda t: (0, 0)),
        ],
        out_specs=pl.BlockSpec((1, BT2, D), lambda t: (0, t, 0)),
        out_shape=jax.ShapeDtypeStruct((1, T, D), jnp.float32),
        compiler_params=pltpu.CompilerParams(
            dimension_semantics=("parallel",),
        ),
    )(
        x2d,
        slot_out,
        gamma2.reshape(1, D),
        beta2.reshape(1, D),
        Wd1.astype(_DOT),
        bd1.reshape(1, H),
        Wd2.astype(_DOT),
        bd2.reshape(1, D),
        Wg.reshape(1, D),
        bg.reshape(1, 1),
    )
    return out
